# baseline (speedup 1.0000x reference)
import jax
import jax.numpy as jnp
from jax.experimental import pallas as pl
from jax.experimental.pallas import tpu as pltpu

H, W, C = 64, 64, 21
N = H * W
THETA_ALPHA, THETA_BETA, THETA_GAMMA = 80.0, 13.0, 3.0
BILATERAL_COMPAT, SPATIAL_COMPAT = 10.0, 3.0
NUM_ITERATIONS = 5

CP = 128
BIG = 1.0e9

RT_A = 256
BT = 512
NI = N // BT
NT_A = N // RT_A
WC = W * CP


def _build_body(fb_ref, fbT_ref, kb_ref, nb_ref, nbb_ref):
    fi = fb_ref[...]
    fT = fbT_ref[...]
    sqi = jnp.sum(fi * fi, axis=1, keepdims=True)
    sqj = jnp.sum(fT * fT, axis=0, keepdims=True)
    cross = jnp.dot(fi, fT, preferred_element_type=jnp.float32)
    d2 = jnp.maximum(sqi + sqj - 2.0 * cross, 0.0)
    k = jnp.exp(-0.5 * d2)
    kb_ref[...] = k.astype(jnp.bfloat16)
    rs = jnp.sum(k, axis=1, keepdims=True)
    nb = 1.0 / (jnp.sqrt(rs) + 1e-20)
    nb_ref[...] = nb
    nbb_ref[...] = jnp.broadcast_to(nb, (RT_A, CP))


def _softmax(x):
    m = jnp.max(x, axis=-1, keepdims=True)
    e = jnp.exp(x - m)
    return e / jnp.sum(e, axis=-1, keepdims=True)


def _iterate_body(u_ref, kb_ref, nb_ref, nbb_ref, out_ref,
                  xb0, xb1, qn0, qn1, sp, g2):
    t = pl.program_id(0)
    i = pl.program_id(1)

    @pl.when((t == 0) & (i == 0))
    def _init():
        ra = jax.lax.broadcasted_iota(jnp.int32, (W, 8), 0)
        ca = jax.lax.broadcasted_iota(jnp.int32, (W, 8), 1)
        gf = jnp.where(ca == 0, ra.astype(jnp.float32) / THETA_GAMMA, 0.0)
        rb = jax.lax.broadcasted_iota(jnp.int32, (8, W), 0)
        cb = jax.lax.broadcasted_iota(jnp.int32, (8, W), 1)
        gfT = jnp.where(rb == 0, cb.astype(jnp.float32) / THETA_GAMMA, 0.0)
        sqa = jnp.sum(gf * gf, axis=1, keepdims=True)
        sqb = jnp.sum(gfT * gfT, axis=0, keepdims=True)
        crossg = jnp.dot(gf, gfT, preferred_element_type=jnp.float32)
        d2g = jnp.maximum(sqa + sqb - 2.0 * crossg, 0.0)
        g = jnp.exp(-0.5 * d2g)
        gs = jnp.sum(g, axis=1, keepdims=True)
        dinv = 1.0 / (jnp.sqrt(gs) + 1e-20)
        scale = SPATIAL_COMPAT ** 0.5
        g2[...] = (scale * dinv) * g * dinv.reshape(1, W)

        q0 = _softmax(-u_ref[...])
        xb0[...] = (q0 * nb_ref[...]).astype(jnp.bfloat16)
        qn0[...] = q0.astype(jnp.bfloat16)

    read0 = (t % 2) == 0
    kb = kb_ref[...]
    rows = pl.ds(i * BT, BT)

    def spatial(qn_src):
        x1 = qn_src[...].astype(jnp.float32).reshape(H, WC)
        t1 = jnp.dot(g2[...], x1, preferred_element_type=jnp.float32)
        t1g = jnp.swapaxes(t1.reshape(H, W, CP), 0, 1)
        t2 = jnp.dot(g2[...], t1g.reshape(W, H * CP),
                     preferred_element_type=jnp.float32)
        spg = jnp.swapaxes(t2.reshape(W, H, CP), 0, 1)
        sp[...] = spg.reshape(N, CP)

    def step(xb_src, qn_src, xb_dst, qn_dst):
        @pl.when(i == 0)
        def _sp():
            spatial(qn_src)

        accb = jnp.dot(kb, xb_src[...], preferred_element_type=jnp.float32)
        nbb = nbb_ref[...]
        msg = (BILATERAL_COMPAT * nbb) * accb + sp[rows, :]
        qnew = _softmax(msg - u_ref[rows, :])
        xb_dst[rows, :] = (qnew * nbb).astype(jnp.bfloat16)
        qn_dst[rows, :] = qnew.astype(jnp.bfloat16)

        @pl.when(t == NUM_ITERATIONS - 1)
        def _out():
            out_ref[rows, :] = qnew

    @pl.when(read0)
    def _step0():
        step(xb0, qn0, xb1, qn1)

    @pl.when(jnp.logical_not(read0))
    def _step1():
        step(xb1, qn1, xb0, qn0)


@jax.jit
def kernel(unary, image):
    f32 = jnp.float32
    ys, xs = jnp.meshgrid(jnp.arange(H, dtype=f32),
                          jnp.arange(W, dtype=f32), indexing="ij")
    zeros1 = jnp.zeros((N, 1), f32)
    fb = jnp.concatenate(
        [(xs / THETA_ALPHA).reshape(N, 1), (ys / THETA_ALPHA).reshape(N, 1),
         (image / THETA_BETA).reshape(N, 3), zeros1, zeros1, zeros1], axis=1)
    fbT = fb.T

    kb, nb, nbb = pl.pallas_call(
        _build_body,
        grid=(NT_A,),
        in_specs=[
            pl.BlockSpec((RT_A, 8), lambda i: (i, 0)),
            pl.BlockSpec((8, N), lambda i: (0, 0)),
        ],
        out_specs=[
            pl.BlockSpec((RT_A, N), lambda i: (i, 0)),
            pl.BlockSpec((RT_A, 1), lambda i: (i, 0)),
            pl.BlockSpec((RT_A, CP), lambda i: (i, 0)),
        ],
        out_shape=[
            jax.ShapeDtypeStruct((N, N), jnp.bfloat16),
            jax.ShapeDtypeStruct((N, 1), f32),
            jax.ShapeDtypeStruct((N, CP), f32),
        ],
    )(fb, fbT)

    u = unary.reshape(N, C)
    u_pad = jnp.full((N, CP), BIG, f32).at[:, :C].set(u)

    q = pl.pallas_call(
        _iterate_body,
        grid=(NUM_ITERATIONS, NI),
        in_specs=[
            pl.BlockSpec((N, CP), lambda t, i: (0, 0)),
            pl.BlockSpec((BT, N), lambda t, i: (i, 0)),
            pl.BlockSpec((N, 1), lambda t, i: (0, 0)),
            pl.BlockSpec((BT, CP), lambda t, i: (i, 0)),
        ],
        out_specs=pl.BlockSpec((N, CP), lambda t, i: (0, 0)),
        out_shape=jax.ShapeDtypeStruct((N, CP), f32),
        scratch_shapes=[
            pltpu.VMEM((N, CP), jnp.bfloat16),
            pltpu.VMEM((N, CP), jnp.bfloat16),
            pltpu.VMEM((N, CP), jnp.bfloat16),
            pltpu.VMEM((N, CP), jnp.bfloat16),
            pltpu.VMEM((N, CP), f32),
            pltpu.VMEM((W, W), f32),
        ],
    )(u_pad, kb, nb, nbb)

    return q[:, :C].reshape(H, W, C)


# speedup vs baseline: 1.0099x; 1.0099x over previous
import jax
import jax.numpy as jnp
from jax.experimental import pallas as pl
from jax.experimental.pallas import tpu as pltpu

H, W, C = 64, 64, 21
N = H * W
THETA_ALPHA, THETA_BETA, THETA_GAMMA = 80.0, 13.0, 3.0
BILATERAL_COMPAT, SPATIAL_COMPAT = 10.0, 3.0
NUM_ITERATIONS = 5

CP = 128
BIG = 1.0e9

RT_A = 256
BT = 512
NI = N // BT
NT_A = N // RT_A
WC = W * CP


def _build_body(fb_ref, fbT_ref, kb_ref, nb_ref, nbb_ref):
    fi = fb_ref[...]
    fT = fbT_ref[...]
    sqi = jnp.sum(fi * fi, axis=1, keepdims=True)
    sqj = jnp.sum(fT * fT, axis=0, keepdims=True)
    cross = jnp.dot(fi, fT, preferred_element_type=jnp.float32)
    d2 = jnp.maximum(sqi + sqj - 2.0 * cross, 0.0)
    k = jnp.exp(-0.5 * d2)
    kb_ref[...] = k.astype(jnp.bfloat16)
    rs = jnp.sum(k, axis=1, keepdims=True)
    nb = 1.0 / (jnp.sqrt(rs) + 1e-20)
    nb_ref[...] = nb
    nbb_ref[...] = jnp.broadcast_to(nb, (RT_A, CP))


def _softmax(x):
    m = jnp.max(x, axis=-1, keepdims=True)
    e = jnp.exp(x - m)
    return e / jnp.sum(e, axis=-1, keepdims=True)


def _iterate_body(u_ref, kb_ref, nb_ref, nbb_ref, out_ref,
                  xb0, xb1, xq0, xq1, sp, g2):
    t = pl.program_id(0)
    i = pl.program_id(1)

    @pl.when((t == 0) & (i == 0))
    def _init():
        ra = jax.lax.broadcasted_iota(jnp.int32, (W, 8), 0)
        ca = jax.lax.broadcasted_iota(jnp.int32, (W, 8), 1)
        gf = jnp.where(ca == 0, ra.astype(jnp.float32) / THETA_GAMMA, 0.0)
        rb = jax.lax.broadcasted_iota(jnp.int32, (8, W), 0)
        cb = jax.lax.broadcasted_iota(jnp.int32, (8, W), 1)
        gfT = jnp.where(rb == 0, cb.astype(jnp.float32) / THETA_GAMMA, 0.0)
        sqa = jnp.sum(gf * gf, axis=1, keepdims=True)
        sqb = jnp.sum(gfT * gfT, axis=0, keepdims=True)
        crossg = jnp.dot(gf, gfT, preferred_element_type=jnp.float32)
        d2g = jnp.maximum(sqa + sqb - 2.0 * crossg, 0.0)
        g = jnp.exp(-0.5 * d2g)
        gs = jnp.sum(g, axis=1, keepdims=True)
        dinv = 1.0 / (jnp.sqrt(gs) + 1e-20)
        scale = SPATIAL_COMPAT ** 0.5
        g2[...] = (scale * dinv) * g * dinv.reshape(1, W)

        q0 = _softmax(-u_ref[...])
        xb0[...] = (q0 * nb_ref[...]).astype(jnp.bfloat16)
        xq0[...] = q0.reshape(H, WC)

    read0 = (t % 2) == 0
    kb = kb_ref[...]
    rows = pl.ds(i * BT, BT)

    def spatial(xq_src):
        x1 = xq_src[...]
        t1 = jnp.dot(g2[...], x1, preferred_element_type=jnp.float32)
        t1g = jnp.swapaxes(t1.reshape(H, W, CP), 0, 1)
        t2 = jnp.dot(g2[...], t1g.reshape(W, H * CP),
                     preferred_element_type=jnp.float32)
        spg = jnp.swapaxes(t2.reshape(W, H, CP), 0, 1)
        sp[...] = spg.reshape(N, CP)

    def step(xb_src, xq_src, xb_dst, xq_dst):
        @pl.when(i == 0)
        def _sp():
            spatial(xq_src)

        accb = jnp.dot(kb, xb_src[...], preferred_element_type=jnp.float32)
        nbb = nbb_ref[...]
        msg = (BILATERAL_COMPAT * nbb) * accb + sp[rows, :]
        qnew = _softmax(msg - u_ref[rows, :])
        xb_dst[rows, :] = (qnew * nbb).astype(jnp.bfloat16)
        xq_dst[pl.ds(i * (BT // W), BT // W), :] = qnew.reshape(BT // W, WC)

        @pl.when(t == NUM_ITERATIONS - 1)
        def _out():
            out_ref[rows, :] = qnew

    @pl.when(read0)
    def _step0():
        step(xb0, xq0, xb1, xq1)

    @pl.when(jnp.logical_not(read0))
    def _step1():
        step(xb1, xq1, xb0, xq0)


@jax.jit
def kernel(unary, image):
    f32 = jnp.float32
    ys, xs = jnp.meshgrid(jnp.arange(H, dtype=f32),
                          jnp.arange(W, dtype=f32), indexing="ij")
    zeros1 = jnp.zeros((N, 1), f32)
    fb = jnp.concatenate(
        [(xs / THETA_ALPHA).reshape(N, 1), (ys / THETA_ALPHA).reshape(N, 1),
         (image / THETA_BETA).reshape(N, 3), zeros1, zeros1, zeros1], axis=1)
    fbT = fb.T

    kb, nb, nbb = pl.pallas_call(
        _build_body,
        grid=(NT_A,),
        in_specs=[
            pl.BlockSpec((RT_A, 8), lambda i: (i, 0)),
            pl.BlockSpec((8, N), lambda i: (0, 0)),
        ],
        out_specs=[
            pl.BlockSpec((RT_A, N), lambda i: (i, 0)),
            pl.BlockSpec((RT_A, 1), lambda i: (i, 0)),
            pl.BlockSpec((RT_A, CP), lambda i: (i, 0)),
        ],
        out_shape=[
            jax.ShapeDtypeStruct((N, N), jnp.bfloat16),
            jax.ShapeDtypeStruct((N, 1), f32),
            jax.ShapeDtypeStruct((N, CP), f32),
        ],
    )(fb, fbT)

    u = unary.reshape(N, C)
    u_pad = jnp.full((N, CP), BIG, f32).at[:, :C].set(u)

    q = pl.pallas_call(
        _iterate_body,
        grid=(NUM_ITERATIONS, NI),
        in_specs=[
            pl.BlockSpec((N, CP), lambda t, i: (0, 0)),
            pl.BlockSpec((BT, N), lambda t, i: (i, 0)),
            pl.BlockSpec((N, 1), lambda t, i: (0, 0)),
            pl.BlockSpec((BT, CP), lambda t, i: (i, 0)),
        ],
        out_specs=pl.BlockSpec((N, CP), lambda t, i: (0, 0)),
        out_shape=jax.ShapeDtypeStruct((N, CP), f32),
        scratch_shapes=[
            pltpu.VMEM((N, CP), jnp.bfloat16),
            pltpu.VMEM((N, CP), jnp.bfloat16),
            pltpu.VMEM((H, WC), jnp.float32),
            pltpu.VMEM((H, WC), jnp.float32),
            pltpu.VMEM((N, CP), f32),
            pltpu.VMEM((W, W), f32),
        ],
    )(u_pad, kb, nb, nbb)

    return q[:, :C].reshape(H, W, C)


# speedup vs baseline: 1.1531x; 1.1418x over previous
import jax
import jax.numpy as jnp
from jax.experimental import pallas as pl
from jax.experimental.pallas import tpu as pltpu

H, W, C = 64, 64, 21
N = H * W
THETA_ALPHA, THETA_BETA, THETA_GAMMA = 80.0, 13.0, 3.0
BILATERAL_COMPAT, SPATIAL_COMPAT = 10.0, 3.0
NUM_ITERATIONS = 5

CP = 128
BIG = 1.0e9

RT_A = 256
BT = 512
NI = N // BT
NT_A = N // RT_A
WC = W * CP


def _build_body(fb_ref, fbT_ref, kb_ref, nb_ref):
    fi = fb_ref[...]
    fT = fbT_ref[...]
    sqi = jnp.sum(fi * fi, axis=1, keepdims=True)
    sqj = jnp.sum(fT * fT, axis=0, keepdims=True)
    cross = jnp.dot(fi, fT, preferred_element_type=jnp.float32)
    d2 = jnp.maximum(sqi + sqj - 2.0 * cross, 0.0)
    k = jnp.exp(-0.5 * d2)
    kb_ref[...] = k.astype(jnp.float8_e4m3fn)
    rs = jnp.sum(k, axis=1, keepdims=True)
    nb_ref[...] = 1.0 / (jnp.sqrt(rs) + 1e-20)


def _softmax(x):
    m = jnp.max(x, axis=-1, keepdims=True)
    e = jnp.exp(x - m)
    return e / jnp.sum(e, axis=-1, keepdims=True)


def _iterate_body(u_ref, kb_ref, nb_ref, out_ref,
                  xb0, xb1, xq0, xq1, sp, g2):
    t = pl.program_id(0)
    i = pl.program_id(1)

    @pl.when((t == 0) & (i == 0))
    def _init():
        ra = jax.lax.broadcasted_iota(jnp.int32, (W, 8), 0)
        ca = jax.lax.broadcasted_iota(jnp.int32, (W, 8), 1)
        gf = jnp.where(ca == 0, ra.astype(jnp.float32) / THETA_GAMMA, 0.0)
        rb = jax.lax.broadcasted_iota(jnp.int32, (8, W), 0)
        cb = jax.lax.broadcasted_iota(jnp.int32, (8, W), 1)
        gfT = jnp.where(rb == 0, cb.astype(jnp.float32) / THETA_GAMMA, 0.0)
        sqa = jnp.sum(gf * gf, axis=1, keepdims=True)
        sqb = jnp.sum(gfT * gfT, axis=0, keepdims=True)
        crossg = jnp.dot(gf, gfT, preferred_element_type=jnp.float32)
        d2g = jnp.maximum(sqa + sqb - 2.0 * crossg, 0.0)
        g = jnp.exp(-0.5 * d2g)
        gs = jnp.sum(g, axis=1, keepdims=True)
        dinv = 1.0 / (jnp.sqrt(gs) + 1e-20)
        scale = SPATIAL_COMPAT ** 0.5
        g2[...] = (scale * dinv) * g * dinv.reshape(1, W)

        q0 = _softmax(-u_ref[...])
        xb0[...] = (q0 * nb_ref[...]).astype(jnp.bfloat16)
        xq0[...] = q0.reshape(H, WC)

    read0 = (t % 2) == 0
    kb = kb_ref[...].astype(jnp.bfloat16)
    rows = pl.ds(i * BT, BT)

    def spatial(xq_src):
        x1 = xq_src[...]
        t1 = jnp.dot(g2[...], x1, preferred_element_type=jnp.float32)
        t1g = jnp.swapaxes(t1.reshape(H, W, CP), 0, 1)
        t2 = jnp.dot(g2[...], t1g.reshape(W, H * CP),
                     preferred_element_type=jnp.float32)
        spg = jnp.swapaxes(t2.reshape(W, H, CP), 0, 1)
        sp[...] = spg.reshape(N, CP)

    def step(xb_src, xq_src, xb_dst, xq_dst):
        @pl.when(i == 0)
        def _sp():
            spatial(xq_src)

        accb = jnp.dot(kb, xb_src[...], preferred_element_type=jnp.float32)
        nb = nb_ref[rows, :]
        msg = (BILATERAL_COMPAT * nb) * accb + sp[rows, :]
        qnew = _softmax(msg - u_ref[rows, :])
        xb_dst[rows, :] = (qnew * nb).astype(jnp.bfloat16)
        xq_dst[pl.ds(i * (BT // W), BT // W), :] = qnew.reshape(BT // W, WC)

        @pl.when(t == NUM_ITERATIONS - 1)
        def _out():
            out_ref[rows, :] = qnew

    @pl.when(read0)
    def _step0():
        step(xb0, xq0, xb1, xq1)

    @pl.when(jnp.logical_not(read0))
    def _step1():
        step(xb1, xq1, xb0, xq0)


@jax.jit
def kernel(unary, image):
    f32 = jnp.float32
    ys, xs = jnp.meshgrid(jnp.arange(H, dtype=f32),
                          jnp.arange(W, dtype=f32), indexing="ij")
    zeros1 = jnp.zeros((N, 1), f32)
    fb = jnp.concatenate(
        [(xs / THETA_ALPHA).reshape(N, 1), (ys / THETA_ALPHA).reshape(N, 1),
         (image / THETA_BETA).reshape(N, 3), zeros1, zeros1, zeros1], axis=1)
    fbT = fb.T

    kb, nb = pl.pallas_call(
        _build_body,
        grid=(NT_A,),
        in_specs=[
            pl.BlockSpec((RT_A, 8), lambda i: (i, 0)),
            pl.BlockSpec((8, N), lambda i: (0, 0)),
        ],
        out_specs=[
            pl.BlockSpec((RT_A, N), lambda i: (i, 0)),
            pl.BlockSpec((RT_A, 1), lambda i: (i, 0)),
        ],
        out_shape=[
            jax.ShapeDtypeStruct((N, N), jnp.float8_e4m3fn),
            jax.ShapeDtypeStruct((N, 1), f32),
        ],
    )(fb, fbT)

    u = unary.reshape(N, C)
    u_pad = jnp.full((N, CP), BIG, f32).at[:, :C].set(u)

    q = pl.pallas_call(
        _iterate_body,
        grid=(NUM_ITERATIONS, NI),
        in_specs=[
            pl.BlockSpec((N, CP), lambda t, i: (0, 0)),
            pl.BlockSpec((BT, N), lambda t, i: (i, 0)),
            pl.BlockSpec((N, 1), lambda t, i: (0, 0)),
        ],
        out_specs=pl.BlockSpec((N, CP), lambda t, i: (0, 0)),
        out_shape=jax.ShapeDtypeStruct((N, CP), f32),
        scratch_shapes=[
            pltpu.VMEM((N, CP), jnp.bfloat16),
            pltpu.VMEM((N, CP), jnp.bfloat16),
            pltpu.VMEM((H, WC), jnp.float32),
            pltpu.VMEM((H, WC), jnp.float32),
            pltpu.VMEM((N, CP), f32),
            pltpu.VMEM((W, W), f32),
        ],
    )(u_pad, kb, nb)

    return q[:, :C].reshape(H, W, C)
